# baseline (speedup 1.0000x reference)
import functools

import jax
import jax.numpy as jnp
from jax import lax
from jax.experimental import pallas as pl
from jax.experimental.pallas import tpu as pltpu
from jax.experimental.pallas import tpu_sc as plsc

EMBED = 64
BATCH = 4096
HIST = 200
B = BATCH * HIST
NC, NS = 2, 16
NW = NC * NS
BPW = B // NW
CH = 128
NCHUNK = BPW // CH

_mesh = plsc.VectorSubcoreMesh(core_axis_name="c", subcore_axis_name="s")


@functools.partial(
    pl.kernel,
    mesh=_mesh,
    out_type=jax.ShapeDtypeStruct((B, EMBED), jnp.float32),
    scratch_types=[
        pltpu.VMEM((NCHUNK, CH), jnp.int32),
        pltpu.VMEM((CH, EMBED), jnp.float32),
        pltpu.SemaphoreType.DMA,
    ],
    compiler_params=pltpu.CompilerParams(use_tc_tiling_on_sc=False),
)
def _gather_kernel(table_hbm, idx_hbm, out_hbm, idx_v, buf, sem):
    wid = lax.axis_index("s") * NC + lax.axis_index("c")
    row0 = wid * NCHUNK
    pltpu.sync_copy(idx_hbm.at[pl.ds(row0, NCHUNK)], idx_v)
    base = wid * BPW

    def step(g, carry):
        pltpu.async_copy(table_hbm.at[idx_v.at[g]], buf, sem).wait()
        pltpu.sync_copy(buf, out_hbm.at[pl.ds(base + g * CH, CH)])
        return carry

    lax.fori_loop(0, NCHUNK, step, 0)


def kernel(indices, table):
    idx = indices.reshape(NW * NCHUNK, CH).astype(jnp.int32)
    out = _gather_kernel(table, idx)
    return out.reshape(BATCH, HIST, EMBED)


# speedup vs baseline: 1.1171x; 1.1171x over previous
import functools

import jax
import jax.numpy as jnp
from jax import lax
from jax.experimental import pallas as pl
from jax.experimental.pallas import tpu as pltpu
from jax.experimental.pallas import tpu_sc as plsc

EMBED = 64
BATCH = 4096
HIST = 200
B = BATCH * HIST
NC, NS = 2, 16
NW = NC * NS
BPW = B // NW
CH = 128
NCHUNK = BPW // CH
K = 4
G = NCHUNK // K
PAIRS = G // 2

_mesh = plsc.VectorSubcoreMesh(core_axis_name="c", subcore_axis_name="s")


@functools.partial(
    pl.kernel,
    mesh=_mesh,
    out_type=jax.ShapeDtypeStruct((B, EMBED), jnp.float32),
    scratch_types=[
        pltpu.VMEM((NCHUNK, CH), jnp.int32),
        pltpu.VMEM((2 * K, CH, EMBED), jnp.float32),
        pltpu.SemaphoreType.DMA,
        pltpu.SemaphoreType.DMA,
        pltpu.SemaphoreType.DMA,
        pltpu.SemaphoreType.DMA,
    ],
    compiler_params=pltpu.CompilerParams(use_tc_tiling_on_sc=False),
)
def _gather_kernel(table_hbm, idx_hbm, out_hbm, idx_v, bufs, g0, g1, w0, w1):
    wid = lax.axis_index("s") * NC + lax.axis_index("c")
    row0 = wid * NCHUNK
    pltpu.sync_copy(idx_hbm.at[pl.ds(row0, NCHUNK)], idx_v)
    base = wid * BPW
    gsem = (g0, g1)
    wsem = (w0, w1)

    def fire_g(o, s):
        for j in range(K):
            pltpu.async_copy(
                table_hbm.at[idx_v.at[o * K + j]], bufs.at[s * K + j], gsem[s]
            )

    def drain_g(s):
        for j in range(K):
            pltpu.make_async_copy(
                table_hbm.at[pl.ds(0, CH)], bufs.at[s * K + j], gsem[s]
            ).wait()

    def fire_w(o, s):
        for j in range(K):
            pltpu.async_copy(
                bufs.at[s * K + j],
                out_hbm.at[pl.ds(base + (o * K + j) * CH, CH)],
                wsem[s],
            )

    def drain_w(s):
        for j in range(K):
            pltpu.make_async_copy(
                bufs.at[s * K + j], out_hbm.at[pl.ds(j * CH, CH)], wsem[s]
            ).wait()

    fire_g(0, 0)
    fire_g(1, 1)
    drain_g(0)
    fire_w(0, 0)
    drain_w(0)
    fire_g(2, 0)
    drain_g(1)
    fire_w(1, 1)

    def pair(o2, carry):
        drain_w(1)
        fire_g(2 * o2 + 1, 1)
        drain_g(0)
        fire_w(2 * o2, 0)
        drain_w(0)
        fire_g(2 * o2 + 2, 0)
        drain_g(1)
        fire_w(2 * o2 + 1, 1)
        return carry

    lax.fori_loop(1, PAIRS - 1, pair, 0)

    drain_w(1)
    fire_g(2 * (PAIRS - 1) + 1, 1)
    drain_g(0)
    fire_w(2 * (PAIRS - 1), 0)
    drain_w(0)
    drain_g(1)
    fire_w(2 * (PAIRS - 1) + 1, 1)
    drain_w(1)


def kernel(indices, table):
    idx = indices.reshape(NW * NCHUNK, CH).astype(jnp.int32)
    out = _gather_kernel(table, idx)
    return out.reshape(BATCH, HIST, EMBED)
